# Initial kernel scaffold; baseline (speedup 1.0000x reference)
import jax
import jax.numpy as jnp
from jax.experimental import pallas as pl


def kernel(x, params, edge_index0, edge_index1, edge_index2, pool_idx1, pool_idx2):
    raise NotImplementedError("write your pallas kernel here")


# speedup vs baseline: 2.9353x; 2.9353x over previous
import functools

import jax
import jax.numpy as jnp
from jax import lax
from jax.experimental import pallas as pl
from jax.experimental.pallas import tpu as pltpu
from jax.experimental.pallas import tpu_sc as plsc

F32 = jnp.float32
BR = 1024
SK = 8
NSUB = 16
NCORE = 2


def _cdiv(a, b):
    return (a + b - 1) // b


def _rup(a, b):
    return _cdiv(a, b) * b


def _accum_stats(st_ref, z, n, co):
    i = pl.program_id(0)
    rows = lax.broadcasted_iota(jnp.int32, (BR, 1), 0) + i * BR
    mask = rows < n
    zm = jnp.where(mask, z, 0.0)
    nb = jnp.minimum(n - i * BR, BR).astype(F32)
    s_b = jnp.sum(zm, 0, keepdims=True)
    m_b = s_b / nb
    ctr = jnp.where(mask, z - m_b, 0.0)
    m2_b = jnp.sum(ctr * ctr, 0, keepdims=True)
    pad = jnp.zeros((6, co), F32)

    @pl.when(i == 0)
    def _():
        st_ref[...] = jnp.concatenate([s_b, m2_b, pad], axis=0)

    @pl.when(i > 0)
    def _():
        s_a = st_ref[0:1, :]
        m2_a = st_ref[1:2, :]
        n_a = (i * BR) * 1.0
        delta = m_b - s_a / n_a
        m2 = m2_a + m2_b + delta * delta * (n_a * nb / (n_a + nb))
        st_ref[...] = jnp.concatenate([s_a + s_b, m2, pad], axis=0)


def _dense(a, wa, b=None, wb=None, add=None, stats=False):
    n, _ = a.shape
    co = wa.shape[1]
    grid = _cdiv(n, BR)
    has_b = b is not None
    has_add = add is not None

    def body(*refs):
        k = 2
        a_ref, wa_ref = refs[0], refs[1]
        acc = jnp.dot(a_ref[...], wa_ref[...], preferred_element_type=F32)
        if has_b:
            acc = acc + jnp.dot(refs[2][...], refs[3][...],
                                preferred_element_type=F32)
            k = 4
        if has_add:
            acc = acc + refs[k][...]
            k += 1
        z_ref = refs[k]
        z_ref[...] = acc
        if stats:
            st_ref = refs[k + 1]
            _accum_stats(st_ref, acc, n, co)

    ins = [a, wa]
    specs = [pl.BlockSpec((BR, a.shape[1]), lambda i: (i, 0)),
             pl.BlockSpec(wa.shape, lambda i: (0, 0))]
    if has_b:
        ins += [b, wb]
        specs += [pl.BlockSpec((BR, b.shape[1]), lambda i: (i, 0)),
                  pl.BlockSpec(wb.shape, lambda i: (0, 0))]
    if has_add:
        ins.append(add)
        specs.append(pl.BlockSpec((BR, co), lambda i: (i, 0)))
    out_shape = [jax.ShapeDtypeStruct((n, co), F32)]
    out_specs = [pl.BlockSpec((BR, co), lambda i: (i, 0))]
    if stats:
        out_shape.append(jax.ShapeDtypeStruct((8, co), F32))
        out_specs.append(pl.BlockSpec((8, co), lambda i: (0, 0)))
    res = pl.pallas_call(
        body, grid=(grid,), in_specs=specs, out_specs=out_specs,
        out_shape=out_shape)(*ins)
    return res if stats else res[0]


def _colstats(z):
    n, co = z.shape
    grid = _cdiv(n, BR)

    def body(z_ref, st_ref):
        _accum_stats(st_ref, z_ref[...], n, co)

    return pl.pallas_call(
        body, grid=(grid,),
        in_specs=[pl.BlockSpec((BR, co), lambda i: (i, 0))],
        out_specs=pl.BlockSpec((8, co), lambda i: (0, 0)),
        out_shape=jax.ShapeDtypeStruct((8, co), F32))(z)


def _bnact(z, st, g, beta, skip=None, act='lrelu'):
    n, co = z.shape
    grid = _cdiv(n, BR)
    has_skip = skip is not None

    def body(*refs):
        st_ref, g_ref, b_ref, z_ref = refs[0], refs[1], refs[2], refs[3]
        o_ref = refs[5] if has_skip else refs[4]
        m = st_ref[0:1, :] / n
        var = st_ref[1:2, :] / n
        y = (z_ref[...] - m) / jnp.sqrt(var + 1e-5) * g_ref[...] + b_ref[...]
        if has_skip:
            y = y + refs[4][...]
        if act == 'relu':
            y = jnp.maximum(y, 0.0)
        elif act == 'lrelu':
            y = jnp.where(y >= 0.0, y, 0.01 * y)
        o_ref[...] = y

    ins = [st, g.reshape(1, co), beta.reshape(1, co), z]
    specs = [pl.BlockSpec((8, co), lambda i: (0, 0)),
             pl.BlockSpec((1, co), lambda i: (0, 0)),
             pl.BlockSpec((1, co), lambda i: (0, 0)),
             pl.BlockSpec((BR, co), lambda i: (i, 0))]
    if has_skip:
        ins.append(skip)
        specs.append(pl.BlockSpec((BR, co), lambda i: (i, 0)))
    return pl.pallas_call(
        body, grid=(grid,), in_specs=specs,
        out_specs=pl.BlockSpec((BR, co), lambda i: (i, 0)),
        out_shape=jax.ShapeDtypeStruct((n, co), F32))(*ins)


def _seg_sum(values, src_pad, dst_pad, n_out):
    nsrc, c = values.shape
    g_tot = c // 16
    g_half = g_tot // 2
    nw = _rup(n_out, 128)
    assert n_out < nw
    epad = src_pad.shape[0]
    nchunks = epad // 128
    cpt = nchunks // NSUB
    n_full = cpt // SK
    assert cpt % SK == 0
    rpt = nw // NSUB
    zbr = 1024

    tbl = values.reshape(nsrc * g_tot, 16)
    src2d = src_pad.reshape(nchunks, 128)
    dst2d = dst_pad.reshape(nchunks, 128)

    mesh = plsc.VectorSubcoreMesh(core_axis_name="c", subcore_axis_name="s",
                                  num_cores=NCORE, num_subcores=NSUB)

    @functools.partial(
        pl.kernel,
        out_type=jax.ShapeDtypeStruct((g_tot, nw, 16), F32),
        mesh=mesh,
        compiler_params=pltpu.CompilerParams(use_tc_tiling_on_sc=False),
        scratch_types=[
            pltpu.VMEM((SK, 128), jnp.int32),
            pltpu.VMEM((SK, 128), jnp.int32),
            pltpu.VMEM((SK, 128, 16), F32),
            pltpu.VMEM((zbr, 16), F32),
            pltpu.VMEM_SHARED((nw, 16), F32),
            pltpu.SemaphoreType.DMA,
        ])
    def seg(tbl_hbm, src_hbm, dst_hbm, out_hbm,
            idx_v, dst_v, rows_v, zb_v, acc_sh, sem):
        cid = lax.axis_index("c")
        sid = lax.axis_index("s")

        def zb_body(r, carry):
            zb_v[r, :] = jnp.zeros((16,), F32)
            return carry

        lax.fori_loop(0, zbr, zb_body, 0)

        chunk0 = sid * cpt
        zbase = sid * rpt

        for sub in range(g_half):
            gid = cid * g_half + sub
            off = 0
            while off < rpt:
                nrow = min(zbr, rpt - off)
                pltpu.sync_copy(zb_v.at[pl.ds(0, nrow)],
                                acc_sh.at[pl.ds(zbase + off, nrow)])
                off += nrow
            plsc.subcore_barrier()

            def do_super(base_chunk):
                pltpu.sync_copy(src_hbm.at[pl.ds(base_chunk, SK)], idx_v)
                pltpu.sync_copy(dst_hbm.at[pl.ds(base_chunk, SK)], dst_v)
                for k in range(SK):
                    for j in range(8):
                        v = idx_v[k, pl.ds(j * 16, 16)]
                        idx_v[k, pl.ds(j * 16, 16)] = v * g_tot + gid
                cps = [pltpu.async_copy(tbl_hbm.at[idx_v.at[k]],
                                        rows_v.at[k], sem)
                       for k in range(SK)]
                for cp in cps:
                    cp.wait()
                for k in range(SK):
                    pltpu.sync_copy(rows_v.at[k], acc_sh.at[dst_v.at[k]],
                                    add=True)

            def body(i, carry):
                do_super(chunk0 + i * SK)
                return carry

            lax.fori_loop(0, n_full, body, 0)
            plsc.subcore_barrier()

            pltpu.sync_copy(acc_sh.at[pl.ds(sid * rpt, rpt)],
                            out_hbm.at[gid, pl.ds(sid * rpt, rpt)])
            if sub + 1 < g_half:
                plsc.subcore_barrier()

    out = seg(tbl, src2d, dst2d)
    return out.transpose(1, 0, 2).reshape(nw, c)[:n_out]


def _up_gather(values, idx_pad, n_out):
    nsrc, c = values.shape
    c2 = c // 2
    npad = idx_pad.shape[0]
    nchunks = npad // 128
    cpt = nchunks // NSUB
    n_full = cpt // SK
    rem = cpt % SK
    assert npad % 16384 == 0 and rem == 0

    tbl = values.reshape(nsrc * 2, c2)
    idx2d = idx_pad.reshape(nchunks, 128)

    mesh = plsc.VectorSubcoreMesh(core_axis_name="c", subcore_axis_name="s",
                                  num_cores=NCORE, num_subcores=NSUB)

    @functools.partial(
        pl.kernel,
        out_type=jax.ShapeDtypeStruct((2, npad, c2), F32),
        mesh=mesh,
        compiler_params=pltpu.CompilerParams(use_tc_tiling_on_sc=False),
        scratch_types=[
            pltpu.VMEM((SK, 128), jnp.int32),
            pltpu.VMEM((SK, 128, c2), F32),
            pltpu.SemaphoreType.DMA,
        ])
    def gat(tbl_hbm, idx_hbm, out_hbm, idx_v, rows_v, sem):
        cid = lax.axis_index("c")
        sid = lax.axis_index("s")
        chunk0 = sid * cpt

        def do_super(base_chunk, nk):
            pltpu.sync_copy(idx_hbm.at[pl.ds(base_chunk, nk)],
                            idx_v.at[pl.ds(0, nk)])
            for k in range(nk):
                for j in range(8):
                    v = idx_v[k, pl.ds(j * 16, 16)]
                    idx_v[k, pl.ds(j * 16, 16)] = v + v + cid
            cps = [pltpu.async_copy(tbl_hbm.at[idx_v.at[k]], rows_v.at[k],
                                    sem) for k in range(nk)]
            for cp in cps:
                cp.wait()
            for k in range(nk):
                pltpu.sync_copy(
                    rows_v.at[k],
                    out_hbm.at[cid, pl.ds((base_chunk + k) * 128, 128)])

        def body(i, carry):
            do_super(chunk0 + i * SK, SK)
            return carry

        lax.fori_loop(0, n_full, body, 0)
        if rem:
            do_super(chunk0 + n_full * SK, rem)

    out = gat(tbl, idx2d)
    return out.transpose(1, 0, 2).reshape(npad, c)[:n_out]


def _pad_edges(ei, n_out):
    e = ei.shape[1]
    epad = _rup(e, 16384)
    junk = n_out
    src = jnp.pad(ei[0], (0, epad - e))
    dst = jnp.pad(ei[1], (0, epad - e), constant_values=junk)
    return src, dst


def _pad_pool(pool_idx, n_out):
    nf = pool_idx.shape[0]
    npad = _rup(nf, 16384)
    junk = n_out
    src = jnp.pad(jnp.arange(nf, dtype=jnp.int32), (0, npad - nf))
    dst = jnp.pad(pool_idx, (0, npad - nf), constant_values=junk)
    return src, dst


def _pad_idx(idx):
    n = idx.shape[0]
    return jnp.pad(idx, (0, _rup(n, 16384) - n))


def _sconv_z(x, ei_pad, p, n):
    src, dst = ei_pad
    agg = _seg_sum(x, src, dst, n)
    return _dense(x, p['Ws'], b=agg, wb=p['Wn'], stats=True)


def _res(x, ei_pad, p, n):
    z1, st1 = _sconv_z(x, ei_pad, p['c1'], n)
    h = _bnact(z1, st1, p['c1']['g'], p['c1']['beta'], act='lrelu')
    z2, st2 = _sconv_z(h, ei_pad, p['c2'], n)
    if 'down' in p:
        zd, std = _dense(x, p['down']['W'], stats=True)
        sc = _bnact(zd, std, p['down']['g'], p['down']['beta'], act=None)
    else:
        sc = x
    return _bnact(z2, st2, p['c2']['g'], p['c2']['beta'], skip=sc,
                  act='lrelu')


def _down(x, pool_pad, p, n_coarse):
    t = _dense(x, p['W'])
    ps, pd = pool_pad
    agg = _seg_sum(t, ps, pd, n_coarse)
    st = _colstats(agg)
    return _bnact(agg, st, p['g'], p['beta'], act='lrelu')


def _up(x, idx_pad, p, n_fine):
    t = _dense(x, p['W'])
    u = _up_gather(t, idx_pad, n_fine)
    st = _colstats(u)
    return _bnact(u, st, p['g'], p['beta'], act='lrelu')


def kernel(x, params, edge_index0, edge_index1, edge_index2,
           pool_idx1, pool_idx2):
    n0 = x.shape[0]
    n1 = pool_idx2.shape[0]
    n2 = n1 // 2

    ei0 = _pad_edges(edge_index0, n0)
    ei1 = _pad_edges(edge_index1, n1)
    p1 = _pad_pool(pool_idx1, n1)

    z0, st0 = _sconv_z(x, ei0, params['stem'], n0)
    x0 = _bnact(z0, st0, params['stem']['g'], params['stem']['beta'],
                act='relu')
    x1 = _down(x0, p1, params['down1'], n1)
    x1 = _res(x1, ei1, params['s1a'], n1)
    x1 = _res(x1, ei1, params['s1b'], n1)
    ei2 = _pad_edges(edge_index2, n2)
    p2 = _pad_pool(pool_idx2, n2)
    x2 = _down(x1, p2, params['down2'], n2)
    x2 = _res(x2, ei2, params['s2a'], n2)
    x2 = _res(x2, ei2, params['s2b'], n2)
    y3u = _up(x2, _pad_idx(pool_idx2), params['up1'], n1)
    y3 = jnp.concatenate([y3u, x1], axis=1)
    y3 = _res(y3, ei1, params['u1a'], n1)
    y3 = _res(y3, ei1, params['u1b'], n1)
    y4u = _up(y3, _pad_idx(pool_idx1), params['up2'], n0)
    y4 = jnp.concatenate([y4u, x0], axis=1)
    y4 = _res(y4, ei0, params['u2a'], n0)
    y4 = _res(y4, ei0, params['u2b'], n0)
    return y4
